# baseline (speedup 1.0000x reference)
import functools

import jax
import jax.numpy as jnp
import numpy as np
from jax.experimental import pallas as pl
from jax.experimental.pallas import tpu as pltpu

N = 8400
E = 42000
NE = 2100
NC = N + NE
NCP = 10752
BM = 840
NB = N // BM
BUD = 10240
BK = 2048


def _mm_kernel(x_ref, w_ref, o_ref):
    o_ref[...] = jnp.dot(x_ref[...].astype(jnp.bfloat16),
                         w_ref[...].astype(jnp.bfloat16),
                         preferred_element_type=jnp.float32)


def _mm(x, w, bm=840, bn=2048):
    m, k = x.shape
    _, n = w.shape
    bn = min(bn, n)
    grid = (m // bm, n // bn)
    return pl.pallas_call(
        _mm_kernel,
        grid=grid,
        in_specs=[pl.BlockSpec((bm, k), lambda i, j: (i, 0)),
                  pl.BlockSpec((k, bn), lambda i, j: (0, j))],
        out_specs=pl.BlockSpec((bm, bn), lambda i, j: (i, j)),
        out_shape=jax.ShapeDtypeStruct((m, n), jnp.float32),
    )(x, w)


def _qp_kernel(q_ref, ket_ref, o_ref):
    o_ref[...] = jnp.dot(q_ref[...], ket_ref[0],
                         preferred_element_type=jnp.float32)[None].astype(
                             o_ref.dtype)


def _qp(q, ket, C, bm=840, bn=1536):
    H = ket.shape[0]
    grid = (H, N // bm, NCP // bn)
    return pl.pallas_call(
        _qp_kernel,
        grid=grid,
        in_specs=[pl.BlockSpec((bm, C), lambda h, i, j: (i, h)),
                  pl.BlockSpec((1, C, bn), lambda h, i, j: (h, 0, j))],
        out_specs=pl.BlockSpec((1, bm, bn), lambda h, i, j: (h, i, j)),
        out_shape=jax.ShapeDtypeStruct((H, N, NCP), jnp.bfloat16),
    )(q, ket)


def _agg_kernel(m_ref, dl_ref, w_ref, s_ref, o_ref, acc_ref, den_ref, *, nkb):
    k = pl.program_id(2)

    @pl.when(k == 0)
    def _init():
        acc_ref[...] = jnp.zeros_like(acc_ref)
        den_ref[...] = jnp.zeros_like(den_ref)

    dl = dl_ref[0]
    wv = w_ref[0, 0]
    rows = jax.lax.broadcasted_iota(jnp.int32, (BM, BK), 0)
    ohw_f = jnp.where(rows == dl, wv, 0.0)
    den_ref[...] += jnp.sum(ohw_f, axis=1, keepdims=True)
    acc_ref[...] += jnp.dot(ohw_f.astype(jnp.bfloat16),
                            m_ref[0].astype(jnp.bfloat16),
                            preferred_element_type=jnp.float32)

    @pl.when(k == nkb - 1)
    def _fin():
        den = den_ref[:, 0:1] * 0.5
        o_ref[...] = jax.nn.relu(
            acc_ref[...] / (den + 1e-16) + s_ref[...]).astype(o_ref.dtype)


def _agg(m, dl, w, s, C, out_dtype):
    H = m.shape[0]
    nkb = BUD // BK
    grid = (H, NB, nkb)
    kern = functools.partial(_agg_kernel, nkb=nkb)
    return pl.pallas_call(
        kern,
        grid=grid,
        in_specs=[
            pl.BlockSpec((1, BK, C), lambda h, i, k: (h, i * nkb + k, 0)),
            pl.BlockSpec((1, 1, BK), lambda h, i, k: (i, 0, k)),
            pl.BlockSpec((1, 1, 1, BK), lambda h, i, k: (h, i, 0, k)),
            pl.BlockSpec((BM, C), lambda h, i, k: (i, h)),
        ],
        out_specs=pl.BlockSpec((BM, C), lambda h, i, k: (i, h)),
        out_shape=jax.ShapeDtypeStruct((N, H * C), out_dtype),
        scratch_shapes=[pltpu.VMEM((BM, C), jnp.float32),
                        pltpu.VMEM((BM, 128), jnp.float32)],
    )(m, dl, w, s)


def _layer(x_bf, Wcat_bf, bcat, e_tab, dst2, col2, bkt, pos, dl3, colpad,
           H, C, out_dtype):
    HC = H * C
    qkvs = _mm(x_bf, Wcat_bf, bn=min(2048, 4 * HC)) + bcat
    q = qkvs[:, :HC].astype(jnp.bfloat16)
    k = qkvs[:, HC:2 * HC]
    v = qkvs[:, 2 * HC:3 * HC]
    s = qkvs[:, 3 * HC:]

    k3 = k.reshape(N, H, C).transpose(1, 0, 2)
    v3 = v.reshape(N, H, C).transpose(1, 0, 2)
    e3 = e_tab.reshape(NE, H, C).transpose(1, 0, 2)
    pad = jnp.zeros((H, NCP - NC, C), jnp.float32)
    kef = jnp.concatenate([k3, e3, pad], axis=1)
    ve = jnp.concatenate([v3, e3, pad], axis=1)

    scale = np.float32(1.0 / np.sqrt(C))
    ket = kef.astype(jnp.bfloat16).transpose(0, 2, 1)
    qp = _qp(q, ket, C)
    qpg = qp[:, dst2, col2].astype(jnp.float32)
    w = jnp.exp((qpg[:, :E] + qpg[:, E:]) * scale)

    w2 = jnp.concatenate([w, w], axis=1)
    wpad = jnp.zeros((H, NB, BUD), jnp.float32).at[:, bkt, pos].set(w2)
    wpad = wpad.reshape(H, NB, 1, BUD)
    ve2d = ve.reshape(H * NCP, C)
    colpad_h = jnp.concatenate([colpad + hh * NCP for hh in range(H)])
    m = ve2d[colpad_h].reshape(H, NB * BUD, C)

    return _agg(m, dl3, wpad, s, C, out_dtype)


def kernel(x, edge_index, edge_features,
           Wq1, bq1, Wk1, bk1, Wv1, bv1, We1, Ws1, bs1,
           Wq2, bq2, Wk2, bk2, Wv2, bv2, We2, Ws2, bs2):
    src = edge_index[0]
    dst = edge_index[1]
    eid = jnp.arange(E, dtype=jnp.int32) % NE

    dst2 = jnp.concatenate([dst, dst])
    col2 = jnp.concatenate([src, N + eid])
    bkt = dst2 // BM
    oh = (bkt[:, None] == jnp.arange(NB, dtype=jnp.int32)[None, :])
    csum = jnp.cumsum(oh.astype(jnp.int32), axis=0)
    pos = jnp.take_along_axis(csum, bkt[:, None], axis=1)[:, 0] - 1
    colpad = jnp.full((NB, BUD), NCP - 1, jnp.int32).at[bkt, pos].set(col2)
    dlpad = jnp.zeros((NB, BUD), jnp.int32).at[bkt, pos].set(dst2 - bkt * BM)
    dl3 = dlpad.reshape(NB, 1, BUD)
    colpad = colpad.reshape(NB * BUD)

    W1 = jnp.concatenate([Wq1, Wk1, Wv1, Ws1], axis=1).astype(jnp.bfloat16)
    b1 = jnp.concatenate([bq1, bk1, bv1, bs1])
    W2 = jnp.concatenate([Wq2, Wk2, Wv2, Ws2], axis=1).astype(jnp.bfloat16)
    b2 = jnp.concatenate([bq2, bk2, bv2, bs2])
    e_tab1 = edge_features @ We1
    e_tab2 = edge_features @ We2

    h = _layer(x.astype(jnp.bfloat16), W1, b1, e_tab1,
               dst2, col2, bkt, pos, dl3, colpad,
               H=2, C=1024, out_dtype=jnp.bfloat16)
    h2 = _layer(h, W2, b2, e_tab2,
                dst2, col2, bkt, pos, dl3, colpad,
                H=1, C=64, out_dtype=jnp.float32)
    return h2.reshape(-1, 420 * 64)


# speedup vs baseline: 1.1188x; 1.1188x over previous
import functools

import jax
import jax.numpy as jnp
import numpy as np
from jax.experimental import pallas as pl
from jax.experimental.pallas import tpu as pltpu

N = 8400
E = 42000
NE = 2100
NC = N + NE
NCP = 10752
BM = 840
NB = N // BM
BUD = 10240
BK = 2048


def _mm_kernel(x_ref, w_ref, o_ref):
    o_ref[...] = jnp.dot(x_ref[...].astype(jnp.bfloat16),
                         w_ref[...].astype(jnp.bfloat16),
                         preferred_element_type=jnp.float32)


def _mm(x, w, bm=840, bn=2048):
    m, k = x.shape
    _, n = w.shape
    bn = min(bn, n)
    grid = (m // bm, n // bn)
    return pl.pallas_call(
        _mm_kernel,
        grid=grid,
        in_specs=[pl.BlockSpec((bm, k), lambda i, j: (i, 0)),
                  pl.BlockSpec((k, bn), lambda i, j: (0, j))],
        out_specs=pl.BlockSpec((bm, bn), lambda i, j: (i, j)),
        out_shape=jax.ShapeDtypeStruct((m, n), jnp.float32),
    )(x, w)


def _qp_kernel(q_ref, ket_ref, o_ref):
    o_ref[...] = jnp.dot(q_ref[...], ket_ref[0],
                         preferred_element_type=jnp.float32)[None].astype(
                             o_ref.dtype)


def _qp(q, ket, C, bm=840, bn=1536):
    H = ket.shape[0]
    grid = (H, N // bm, NCP // bn)
    return pl.pallas_call(
        _qp_kernel,
        grid=grid,
        in_specs=[pl.BlockSpec((bm, C), lambda h, i, j: (i, h)),
                  pl.BlockSpec((1, C, bn), lambda h, i, j: (h, 0, j))],
        out_specs=pl.BlockSpec((1, bm, bn), lambda h, i, j: (h, i, j)),
        out_shape=jax.ShapeDtypeStruct((H, N, NCP), jnp.float32),
    )(q, ket)


def _agg_kernel(m_ref, dl_ref, w_ref, s_ref, o_ref, acc_ref, den_ref, *, nkb):
    k = pl.program_id(2)

    @pl.when(k == 0)
    def _init():
        acc_ref[...] = jnp.zeros_like(acc_ref)
        den_ref[...] = jnp.zeros_like(den_ref)

    dl = dl_ref[0]
    wv = w_ref[0, 0]
    rows = jax.lax.broadcasted_iota(jnp.int32, (BM, BK), 0)
    ohw_f = jnp.where(rows == dl, wv, 0.0)
    den_ref[...] += jnp.sum(ohw_f, axis=1, keepdims=True)
    acc_ref[...] += jnp.dot(ohw_f.astype(jnp.bfloat16),
                            m_ref[0].astype(jnp.bfloat16),
                            preferred_element_type=jnp.float32)

    @pl.when(k == nkb - 1)
    def _fin():
        den = den_ref[:, 0:1] * 0.5
        o_ref[...] = jax.nn.relu(
            acc_ref[...] / (den + 1e-16) + s_ref[...]).astype(o_ref.dtype)


def _agg(m, dl, w, s, C, out_dtype):
    H = m.shape[0]
    nkb = BUD // BK
    grid = (H, NB, nkb)
    kern = functools.partial(_agg_kernel, nkb=nkb)
    return pl.pallas_call(
        kern,
        grid=grid,
        in_specs=[
            pl.BlockSpec((1, BK, C), lambda h, i, k: (h, i * nkb + k, 0)),
            pl.BlockSpec((1, 1, BK), lambda h, i, k: (i, 0, k)),
            pl.BlockSpec((1, 1, 1, BK), lambda h, i, k: (h, i, 0, k)),
            pl.BlockSpec((BM, C), lambda h, i, k: (i, h)),
        ],
        out_specs=pl.BlockSpec((BM, C), lambda h, i, k: (i, h)),
        out_shape=jax.ShapeDtypeStruct((N, H * C), out_dtype),
        scratch_shapes=[pltpu.VMEM((BM, C), jnp.float32),
                        pltpu.VMEM((BM, 128), jnp.float32)],
    )(m, dl, w, s)


def _layer(x_bf, Wcat_bf, bcat, e_tab, dst2, col2, bkt, pos, dl3, colpad,
           H, C, out_dtype):
    HC = H * C
    qkvs = _mm(x_bf, Wcat_bf, bn=min(2048, 4 * HC)) + bcat
    q = qkvs[:, :HC].astype(jnp.bfloat16)
    k = qkvs[:, HC:2 * HC]
    v = qkvs[:, 2 * HC:3 * HC]
    s = qkvs[:, 3 * HC:]

    k3 = k.reshape(N, H, C).transpose(1, 0, 2)
    v3 = v.reshape(N, H, C).transpose(1, 0, 2)
    e3 = e_tab.reshape(NE, H, C).transpose(1, 0, 2)
    pad = jnp.zeros((H, NCP - NC, C), jnp.float32)
    kef = jnp.concatenate([k3, e3, pad], axis=1)
    ve = jnp.concatenate([v3, e3, pad], axis=1)

    scale = np.float32(1.0 / np.sqrt(C))
    ket = kef.astype(jnp.bfloat16).transpose(0, 2, 1)
    qp = _qp(q, ket, C)
    qpg = qp[:, dst2, col2]
    w = jnp.exp((qpg[:, :E] + qpg[:, E:]) * scale)

    w2 = jnp.concatenate([w, w], axis=1)
    wpad = jnp.zeros((H, NB, BUD), jnp.float32).at[:, bkt, pos].set(w2)
    wpad = wpad.reshape(H, NB, 1, BUD)
    ve2d = ve.reshape(H * NCP, C)
    colpad_h = jnp.concatenate([colpad + hh * NCP for hh in range(H)])
    m = ve2d[colpad_h].reshape(H, NB * BUD, C)

    return _agg(m, dl3, wpad, s, C, out_dtype)


def kernel(x, edge_index, edge_features,
           Wq1, bq1, Wk1, bk1, Wv1, bv1, We1, Ws1, bs1,
           Wq2, bq2, Wk2, bk2, Wv2, bv2, We2, Ws2, bs2):
    src = edge_index[0]
    dst = edge_index[1]
    eid = jnp.arange(E, dtype=jnp.int32) % NE

    dst2 = jnp.concatenate([dst, dst])
    col2 = jnp.concatenate([src, N + eid])
    bkt = dst2 // BM
    oh = (bkt[:, None] == jnp.arange(NB, dtype=jnp.int32)[None, :])
    csum = jnp.cumsum(oh.astype(jnp.int32), axis=0)
    pos = jnp.take_along_axis(csum, bkt[:, None], axis=1)[:, 0] - 1
    colpad = jnp.full((NB, BUD), NCP - 1, jnp.int32).at[bkt, pos].set(col2)
    dlpad = jnp.zeros((NB, BUD), jnp.int32).at[bkt, pos].set(dst2 - bkt * BM)
    dl3 = dlpad.reshape(NB, 1, BUD)
    colpad = colpad.reshape(NB * BUD)

    W1 = jnp.concatenate([Wq1, Wk1, Wv1, Ws1], axis=1).astype(jnp.bfloat16)
    b1 = jnp.concatenate([bq1, bk1, bv1, bs1])
    W2 = jnp.concatenate([Wq2, Wk2, Wv2, Ws2], axis=1).astype(jnp.bfloat16)
    b2 = jnp.concatenate([bq2, bk2, bv2, bs2])
    e_tab1 = edge_features @ We1
    e_tab2 = edge_features @ We2

    h = _layer(x.astype(jnp.bfloat16), W1, b1, e_tab1,
               dst2, col2, bkt, pos, dl3, colpad,
               H=2, C=1024, out_dtype=jnp.bfloat16)
    h2 = _layer(h, W2, b2, e_tab2,
                dst2, col2, bkt, pos, dl3, colpad,
                H=1, C=64, out_dtype=jnp.float32)
    return h2.reshape(-1, 420 * 64)
